# baseline (speedup 1.0000x reference)
import jax
import jax.numpy as jnp
from jax.experimental import pallas as pl

_TEMP = 1.0
_EPS = 1e-10


def _gumbel_softmax_kernel(x_ref, g_ref, o_ref):
    p = x_ref[...] + g_ref[...]
    m = jnp.max(p, axis=1, keepdims=True)
    e = jnp.exp(p - m)
    s = jnp.sum(e, axis=1, keepdims=True)
    o_ref[...] = e / s


def kernel(logits):
    rows, cols = logits.shape
    u = jax.random.uniform(jax.random.key(1), logits.shape, logits.dtype)
    g = -jnp.log(_EPS - jnp.log(u + _EPS))
    block_rows = 8
    return pl.pallas_call(
        _gumbel_softmax_kernel,
        grid=(rows // block_rows,),
        in_specs=[
            pl.BlockSpec((block_rows, cols), lambda i: (i, 0)),
            pl.BlockSpec((block_rows, cols), lambda i: (i, 0)),
        ],
        out_specs=pl.BlockSpec((block_rows, cols), lambda i: (i, 0)),
        out_shape=jax.ShapeDtypeStruct((rows, cols), logits.dtype),
    )(logits, g)
